# Initial kernel scaffold; baseline (speedup 1.0000x reference)
import jax
import jax.numpy as jnp
from jax.experimental import pallas as pl


def kernel(tm_w1, tm_b1, tm_w2, tm_b2, rm_w1, rm_b1, rm_w2, rm_b2, rm_w3, rm_b3, d0r1_w0, d0r1_w1, d0r1_tw, d0r1_pv, d0r1_gavg, d0r1_wr, d0r2_w0, d0r2_w1, d0r2_tw, d0r2_pv, d0r2_gavg, d0_dw, d0_db, d1r1_w0, d1r1_w1, d1r1_tw, d1r1_pv, d1r1_gavg, d1r1_wr, d1r2_w0, d1r2_w1, d1r2_tw, d1r2_pv, d1r2_gavg, m1_w0, m1_w1, m1_tw, m1_pv, m1_gavg, m2_w0, m2_w1, m2_tw, m2_pv, m2_gavg, u0r1_w0, u0r1_w1, u0r1_tw, u0r1_pv, u0r1_gavg, u0r1_wr, u0r2_w0, u0r2_w1, u0r2_tw, u0r2_pv, u0r2_gavg, u0_uw, u0_ub, f_w0, f_pv, f_gavg, f_wf, f_bf, x, time, returns):
    raise NotImplementedError("write your pallas kernel here")


# speedup vs baseline: 1.3985x; 1.3985x over previous
import functools
import math

import jax
import jax.numpy as jnp
import numpy as np
from jax import lax
from jax.experimental import pallas as pl
from jax.experimental.pallas import tpu as pltpu

_EPS = 1e-5
_N = 2
_H0 = 256
_H1 = 128
_K = 5

_BF = jnp.bfloat16


def _softplus_tanh(y):
    t2 = jnp.square(1.0 + jnp.exp(jnp.minimum(y, 20.0)))
    return jnp.where(y > 20.0, 1.0, (t2 - 1.0) / (t2 + 1.0))


def _im2col(x, K, H):
    cin = x.shape[0]
    pad = K // 2
    xb = x.astype(_BF)
    zer = jnp.zeros((cin, pad), _BF)
    xp = jnp.concatenate([zer, xb, zer], axis=1)
    return jnp.concatenate([xp[:, t:t + H] for t in range(K)], axis=0)


def _cgm(x, w, b, g, beta, gavg, *, K, H, n):
    cols = jnp.concatenate(
        [_im2col(x[:, e * H:(e + 1) * H], K, H) for e in range(n)], axis=1)
    acc = jnp.dot(w, cols, preferred_element_type=jnp.float32) + b

    inv_h = 1.0 / float(H)
    cout = acc.shape[0]
    stats = jnp.concatenate(
        [jnp.concatenate(
            [jnp.sum(acc[:, e * H:(e + 1) * H], axis=1, keepdims=True),
             jnp.sum(jnp.square(acc[:, e * H:(e + 1) * H]), axis=1,
                     keepdims=True)], axis=1)
         for e in range(n)], axis=1) * inv_h
    gs = jnp.dot(gavg, stats, preferred_element_type=jnp.float32)
    scs, shs = [], []
    for e in range(n):
        mean = gs[:, 2 * e:2 * e + 1]
        var = jnp.maximum(gs[:, 2 * e + 1:2 * e + 2] - mean * mean, 0.0)
        sc = g * lax.rsqrt(var + _EPS)
        scs.append(jnp.broadcast_to(sc, (cout, H)))
        shs.append(jnp.broadcast_to(beta - mean * sc, (cout, H)))
    y = acc * jnp.concatenate(scs, axis=1) + jnp.concatenate(shs, axis=1)
    return y * _softplus_tanh(y)


def _rb(x, mtb, w0, w1, tw, pv, gavg, wr, *, K, H, n):
    cout = pv.shape[0]
    tb = jnp.dot(tw, mtb, preferred_element_type=jnp.float32) + pv[:, 7:8]
    tbx = jnp.concatenate(
        [jnp.broadcast_to(tb[:, e:e + 1], (cout, H)) for e in range(n)], axis=1)
    h = _cgm(x, w0, pv[:, 0:1], pv[:, 1:2], pv[:, 2:3], gavg,
             K=K, H=H, n=n) + tbx
    y = _cgm(h, w1, pv[:, 3:4], pv[:, 4:5], pv[:, 5:6], gavg, K=K, H=H, n=n)
    if wr is None:
        res = x
    else:
        res = jnp.dot(wr, x.astype(_BF),
                      preferred_element_type=jnp.float32) + pv[:, 6:7]
    return y + res


def _unet_kernel(x_ref, mt_ref,
                 a_w0, a_w1, a_tw, a_pv, a_wr,
                 b_w0, b_w1, b_tw, b_pv,
                 g256, g512,
                 dwd, dsev, ddb,
                 c_w0, c_w1, c_tw, c_pv, c_wr,
                 d_w0, d_w1, d_tw, d_pv,
                 e_w0, e_w1, e_tw, e_pv,
                 f_w0_, f_w1, f_tw, f_pv,
                 p_w0, p_w1, p_tw, p_pv, p_wr,
                 q_w0, q_w1, q_tw, q_pv,
                 uwe, uwo, upe, upo, uub,
                 z_w0, z_pv, z_wf, z_bf,
                 o_ref, *, n):
    H0, H1, K = _H0, _H1, _K
    ga = g256[...]
    gb = g512[...]

    x = jnp.concatenate([x_ref[e].astype(jnp.float32) for e in range(n)],
                        axis=1)
    mtb = jnp.concatenate([mt_ref[e] for e in range(n)],
                          axis=1).astype(_BF)

    x = _rb(x, mtb, a_w0[...], a_w1[...], a_tw[...], a_pv[...], ga,
            a_wr[...], K=K, H=H0, n=n)
    x = _rb(x, mtb, b_w0[...], b_w1[...], b_tw[...], b_pv[...], ga,
            None, K=K, H=H0, n=n)

    parts = []
    for e in range(n):
        xe = x[:, e * H0:(e + 1) * H0].astype(_BF)
        z1 = jnp.zeros((xe.shape[0], 1), _BF)
        xp = jnp.concatenate([z1, xe, z1], axis=1)
        big = jnp.concatenate([xp[:, t:t + H0] for t in range(3)], axis=0)
        ce = jnp.dot(big, dsev[...], preferred_element_type=jnp.float32)
        parts.append(jnp.dot(dwd[...], ce.astype(_BF),
                             preferred_element_type=jnp.float32))
    x = jnp.concatenate(parts, axis=1) + ddb[...]

    x = _rb(x, mtb, c_w0[...], c_w1[...], c_tw[...], c_pv[...], gb,
            c_wr[...], K=K, H=H1, n=n)
    x = _rb(x, mtb, d_w0[...], d_w1[...], d_tw[...], d_pv[...], gb,
            None, K=K, H=H1, n=n)
    skip = x

    x = _rb(x, mtb, e_w0[...], e_w1[...], e_tw[...], e_pv[...], gb,
            None, K=K, H=H1, n=n)
    x = _rb(x, mtb, f_w0_[...], f_w1[...], f_tw[...], f_pv[...], gb,
            None, K=K, H=H1, n=n)

    x = jnp.concatenate([x, skip], axis=0)
    x = _rb(x, mtb, p_w0[...], p_w1[...], p_tw[...], p_pv[...], ga,
            p_wr[...], K=K, H=H1, n=n)
    x = _rb(x, mtb, q_w0[...], q_w1[...], q_tw[...], q_pv[...], ga,
            None, K=K, H=H1, n=n)

    parts = []
    for e in range(n):
        xe = x[:, e * H1:(e + 1) * H1].astype(_BF)
        z1 = jnp.zeros((xe.shape[0], 1), _BF)
        xm1 = jnp.concatenate([z1, xe[:, :H1 - 1]], axis=1)
        xp1 = jnp.concatenate([xe[:, 1:], z1], axis=1)
        ev = jnp.dot(uwe[...], jnp.concatenate([xm1, xe], axis=0),
                     preferred_element_type=jnp.float32)
        od = jnp.dot(uwo[...], jnp.concatenate([xe, xp1], axis=0),
                     preferred_element_type=jnp.float32)
        parts.append(
            jnp.dot(ev.astype(_BF), upe[...],
                    preferred_element_type=jnp.float32)
            + jnp.dot(od.astype(_BF), upo[...],
                      preferred_element_type=jnp.float32))
    x = jnp.concatenate(parts, axis=1) + uub[...]

    pvz = z_pv[...]
    y = _cgm(x, z_w0[...], pvz[:, 0:1], pvz[:, 1:2], pvz[:, 2:3], ga,
             K=K, H=H0, n=n)
    out = jnp.dot(z_wf[...], y.astype(_BF),
                  preferred_element_type=jnp.float32) + z_bf[...]
    for e in range(n):
        o_ref[e] = out[:, e * H0:(e + 1) * H0]


def _mish(v):
    return v * jnp.tanh(jax.nn.softplus(v))


def _sin_emb(t, dim):
    half = dim // 2
    freq = jnp.exp(jnp.arange(half, dtype=jnp.float32)
                   * (-math.log(10000.0) / (half - 1)))
    args = t.astype(jnp.float32)[:, None] * freq[None, :]
    return jnp.concatenate([jnp.sin(args), jnp.cos(args)], axis=-1)


def _cspec(shape):
    nd = len(shape)
    return pl.BlockSpec(shape, lambda i: (0,) * nd)


@functools.lru_cache(maxsize=None)
def _sel_matrices():
    sev = np.zeros((_H0, _H1), np.float32)
    sev[2 * np.arange(_H1), np.arange(_H1)] = 1.0
    pe = np.zeros((_H1, _H0), np.float32)
    po = np.zeros((_H1, _H0), np.float32)
    pe[np.arange(_H1), 2 * np.arange(_H1)] = 1.0
    po[np.arange(_H1), 2 * np.arange(_H1) + 1] = 1.0
    return sev, pe, po


def kernel(tm_w1, tm_b1, tm_w2, tm_b2, rm_w1, rm_b1, rm_w2, rm_b2, rm_w3,
           rm_b3, d0r1_w0, d0r1_w1, d0r1_tw, d0r1_pv, d0r1_gavg, d0r1_wr,
           d0r2_w0, d0r2_w1, d0r2_tw, d0r2_pv, d0r2_gavg, d0_dw, d0_db,
           d1r1_w0, d1r1_w1, d1r1_tw, d1r1_pv, d1r1_gavg, d1r1_wr,
           d1r2_w0, d1r2_w1, d1r2_tw, d1r2_pv, d1r2_gavg,
           m1_w0, m1_w1, m1_tw, m1_pv, m1_gavg,
           m2_w0, m2_w1, m2_tw, m2_pv, m2_gavg,
           u0r1_w0, u0r1_w1, u0r1_tw, u0r1_pv, u0r1_gavg, u0r1_wr,
           u0r2_w0, u0r2_w1, u0r2_tw, u0r2_pv, u0r2_gavg,
           u0_uw, u0_ub, f_w0, f_pv, f_gavg, f_wf, f_bf, x, time, returns):
    B = x.shape[0]
    n = _N
    dim = 256

    e = _sin_emb(time, dim)
    e = _mish(jnp.dot(e, tm_w1.T) + tm_b1)
    t = jnp.dot(e, tm_w2.T) + tm_b2
    r = _mish(jnp.dot(returns, rm_w1.T) + rm_b1)
    r = _mish(jnp.dot(r, rm_w2.T) + rm_b2)
    r = jnp.dot(r, rm_w3.T) + rm_b3
    mt = _mish(jnp.concatenate([t, r], axis=-1))
    mt = mt.reshape(B, 512, 1)

    xt = jnp.transpose(x, (0, 2, 1))

    bf = lambda a: a.astype(_BF)

    cd = d0_dw.shape[0]
    dwd = bf(jnp.transpose(d0_dw, (0, 2, 1)).reshape(cd, 3 * cd))
    ddb = d0_db.reshape(cd, 1)

    cu = u0_uw.shape[0]
    uwe = bf(jnp.concatenate([u0_uw[:, :, 0], u0_uw[:, :, 2]], axis=1))
    uwo = bf(jnp.concatenate([u0_uw[:, :, 1], u0_uw[:, :, 3]], axis=1))
    uub = u0_ub.reshape(cu, 1)

    sev_np, pe_np, po_np = _sel_matrices()
    dsev = jnp.asarray(sev_np, _BF)
    upe = jnp.asarray(pe_np, _BF)
    upo = jnp.asarray(po_np, _BF)

    consts = [
        (bf(d0r1_w0), bf(d0r1_w1), bf(d0r1_tw), d0r1_pv, bf(d0r1_wr)),
        (bf(d0r2_w0), bf(d0r2_w1), bf(d0r2_tw), d0r2_pv),
        (d0r1_gavg, d1r1_gavg),
        (dwd, dsev, ddb),
        (bf(d1r1_w0), bf(d1r1_w1), bf(d1r1_tw), d1r1_pv, bf(d1r1_wr)),
        (bf(d1r2_w0), bf(d1r2_w1), bf(d1r2_tw), d1r2_pv),
        (bf(m1_w0), bf(m1_w1), bf(m1_tw), m1_pv),
        (bf(m2_w0), bf(m2_w1), bf(m2_tw), m2_pv),
        (bf(u0r1_w0), bf(u0r1_w1), bf(u0r1_tw), u0r1_pv, bf(u0r1_wr)),
        (bf(u0r2_w0), bf(u0r2_w1), bf(u0r2_tw), u0r2_pv),
        (uwe, uwo, upe, upo, uub),
        (bf(f_w0), f_pv, bf(f_wf), f_bf),
    ]
    flat = [a for grp in consts for a in grp]

    in_specs = [
        pl.BlockSpec((n, 32, _H0), lambda i: (i, 0, 0)),
        pl.BlockSpec((n, 512, 1), lambda i: (i, 0, 0)),
    ] + [_cspec(a.shape) for a in flat]

    out = pl.pallas_call(
        functools.partial(_unet_kernel, n=n),
        out_shape=jax.ShapeDtypeStruct((B, 32, _H0), x.dtype),
        grid=(B // n,),
        in_specs=in_specs,
        out_specs=pl.BlockSpec((n, 32, _H0), lambda i: (i, 0, 0)),
        compiler_params=pltpu.CompilerParams(
            dimension_semantics=("parallel",),
            vmem_limit_bytes=64 * 1024 * 1024,
        ),
    )(xt, mt, *flat)

    return jnp.transpose(out, (0, 2, 1))
